# baseline (speedup 1.0000x reference)
import jax
import jax.numpy as jnp
from jax.experimental import pallas as pl
from jax.experimental.pallas import tpu as pltpu

_BK = 64
_NBUF = 8


def _body(pos_ref, x_hbm, o_hbm, xbuf, obuf, insem, outsem):
    nchunks = x_hbm.shape[0] // _BK
    pos = pos_ref[...][None]

    def in_copy(chunk, slot):
        return pltpu.make_async_copy(
            x_hbm.at[pl.ds(chunk * _BK, _BK)], xbuf.at[slot], insem.at[slot])

    def out_copy(chunk, slot):
        return pltpu.make_async_copy(
            obuf.at[slot], o_hbm.at[pl.ds(chunk * _BK, _BK)], outsem.at[slot])

    for k in range(_NBUF):
        in_copy(k, k).start()

    def step(i, carry):
        slot = jax.lax.rem(i, _NBUF)
        in_copy(i, slot).wait()

        @pl.when(i >= _NBUF)
        def _():
            out_copy(i - _NBUF, slot).wait()

        obuf[slot] = xbuf[slot] + pos
        out_copy(i, slot).start()

        @pl.when(i + _NBUF < nchunks)
        def _():
            in_copy(i + _NBUF, slot).start()

        return carry

    jax.lax.fori_loop(0, nchunks, step, 0)

    for k in range(_NBUF):
        out_copy(nchunks - _NBUF + k, jax.lax.rem(nchunks - _NBUF + k, _NBUF)).wait()


def kernel(patch, pos_emb):
    B, P, D = patch.shape
    return pl.pallas_call(
        _body,
        in_specs=[
            pl.BlockSpec((P, D), lambda: (0, 0)),
            pl.BlockSpec(memory_space=pl.ANY),
        ],
        out_specs=pl.BlockSpec(memory_space=pl.ANY),
        out_shape=jax.ShapeDtypeStruct((B, P, D), patch.dtype),
        scratch_shapes=[
            pltpu.VMEM((_NBUF, _BK, P, D), patch.dtype),
            pltpu.VMEM((_NBUF, _BK, P, D), patch.dtype),
            pltpu.SemaphoreType.DMA((_NBUF,)),
            pltpu.SemaphoreType.DMA((_NBUF,)),
        ],
    )(pos_emb, patch)


# speedup vs baseline: 3.9984x; 3.9984x over previous
import jax
import jax.numpy as jnp
from jax.experimental import pallas as pl


def _add_body(pos_ref, x_ref, o_ref):
    o_ref[...] = x_ref[...] + pos_ref[...]


def kernel(patch, pos_emb):
    B, P, D = patch.shape
    xt = jnp.transpose(patch, (1, 0, 2))
    pos3 = pos_emb[:, None, :]
    BK = 512
    out = pl.pallas_call(
        _add_body,
        grid=(P, B // BK),
        in_specs=[
            pl.BlockSpec((1, 1, D), lambda p, i: (p, 0, 0)),
            pl.BlockSpec((1, BK, D), lambda p, i: (p, i, 0)),
        ],
        out_specs=pl.BlockSpec((1, BK, D), lambda p, i: (p, i, 0)),
        out_shape=jax.ShapeDtypeStruct((P, B, D), patch.dtype),
    )(pos3, xt)
    return jnp.transpose(out, (1, 0, 2))


# speedup vs baseline: 4.4373x; 1.1098x over previous
import jax
import jax.numpy as jnp
from jax.experimental import pallas as pl
from jax.experimental.pallas import tpu as pltpu

_BK = 256
_NBUF = 8


def _body(pos_ref, x_hbm, o_hbm, xbuf, obuf, insem, outsem):
    P, B, D = x_hbm.shape
    per_plane = B // _BK
    nchunks = P * per_plane

    def in_copy(chunk, slot):
        p = jax.lax.div(chunk, per_plane)
        i = jax.lax.rem(chunk, per_plane)
        return pltpu.make_async_copy(
            x_hbm.at[p, pl.ds(i * _BK, _BK)], xbuf.at[slot], insem.at[slot])

    def out_copy(chunk, slot):
        p = jax.lax.div(chunk, per_plane)
        i = jax.lax.rem(chunk, per_plane)
        return pltpu.make_async_copy(
            obuf.at[slot], o_hbm.at[p, pl.ds(i * _BK, _BK)], outsem.at[slot])

    for k in range(_NBUF):
        in_copy(jnp.int32(k), k).start()

    def step(c, carry):
        slot = jax.lax.rem(c, _NBUF)
        p = jax.lax.div(c, per_plane)
        in_copy(c, slot).wait()

        @pl.when(c >= _NBUF)
        def _():
            out_copy(c - _NBUF, slot).wait()

        obuf[slot] = xbuf[slot] + pos_ref[pl.ds(p, 1)]
        out_copy(c, slot).start()

        @pl.when(c + _NBUF < nchunks)
        def _():
            in_copy(c + _NBUF, slot).start()

        return carry

    jax.lax.fori_loop(0, nchunks, step, 0)

    for k in range(_NBUF):
        c = nchunks - _NBUF + k
        out_copy(jnp.int32(c), c % _NBUF).wait()


def kernel(patch, pos_emb):
    B, P, D = patch.shape
    xt = jnp.transpose(patch, (1, 0, 2))
    out = pl.pallas_call(
        _body,
        in_specs=[
            pl.BlockSpec((P, D), lambda: (0, 0)),
            pl.BlockSpec(memory_space=pl.ANY),
        ],
        out_specs=pl.BlockSpec(memory_space=pl.ANY),
        out_shape=jax.ShapeDtypeStruct((P, B, D), patch.dtype),
        scratch_shapes=[
            pltpu.VMEM((_NBUF, _BK, D), patch.dtype),
            pltpu.VMEM((_NBUF, _BK, D), patch.dtype),
            pltpu.SemaphoreType.DMA((_NBUF,)),
            pltpu.SemaphoreType.DMA((_NBUF,)),
        ],
    )(pos_emb, xt)
    return jnp.transpose(out, (1, 0, 2))
